# baseline (speedup 1.0000x reference)
import jax
import jax.numpy as jnp
from jax import lax
from jax.experimental import pallas as pl
from jax.experimental.pallas import tpu as pltpu
from jax.experimental.pallas import tpu_sc as plsc

_BATCH, _NP, _D = 64, 576, 768
_NC, _NS = 2, 16
_NW = _NC * _NS
_NG = 4
_NB = 8
_BPG = _BATCH // _NG
_BAND = _NP // _NB
_C = 24
_SPB = _BAND // _C
_TCH = _BPG * _SPB
_VPR = _D // 16
_NBUF = 4


def _sc_body(flat_hbm, pos_hbm, out_hbm, posband,
             b0, b1, b2, b3, si0, si1, si2, si3, so0, so1, so2, so3, sst):
    c = lax.axis_index("c")
    s = lax.axis_index("s")
    w = c * _NS + s
    g = w // _NB
    pb = w % _NB
    bufs = (b0, b1, b2, b3)
    sin = (si0, si1, si2, si3)
    sout = (so0, so1, so2, so3)

    stage = pltpu.async_copy(pos_hbm.at[pl.ds(pb * _BAND, _BAND)], posband, sst)

    def row0_of(t):
        b = g * _BPG + t // _SPB
        sub = t % _SPB
        return b * _NP + pb * _BAND + sub * _C

    def start_in(t, j):
        pltpu.async_copy(flat_hbm.at[pl.ds(row0_of(t), _C)], bufs[j], sin[j])

    def wait_in(j):
        pltpu.make_async_copy(flat_hbm.at[pl.ds(0, _C)], bufs[j], sin[j]).wait()

    def start_out(t, j):
        r0 = row0_of(t)
        pltpu.async_copy(bufs[j].at[pl.ds(0, 16)], out_hbm.at[pl.ds(r0, 16)],
                         sout[j])
        pltpu.async_copy(bufs[j].at[pl.ds(16, 8)],
                         out_hbm.at[pl.ds(r0 + 16, 8)], sout[j])

    def wait_out(j):
        pltpu.make_async_copy(bufs[j].at[pl.ds(0, 16)],
                              out_hbm.at[pl.ds(0, 16)], sout[j]).wait()
        pltpu.make_async_copy(bufs[j].at[pl.ds(16, 8)],
                              out_hbm.at[pl.ds(16, 8)], sout[j]).wait()

    start_in(0, 0)
    start_in(1, 1)
    start_in(2, 2)
    stage.wait()

    def outer(g2, carry):
        for j in range(_NBUF):
            t = g2 * _NBUF + j
            wait_in(j)
            prow = (t % _SPB) * _C

            @plsc.parallel_loop(0, _C)
            def row_add(i):
                for k in range(_VPR):
                    plsc.addupdate(bufs[j].at[i, pl.ds(k * 16, 16)],
                                   posband[prow + i, pl.ds(k * 16, 16)])
            start_out(t, j)
            nj = (j + 3) % _NBUF

            @pl.when(t + 3 < _TCH)
            def _prefetch():
                @pl.when(t >= 1)
                def _drain():
                    wait_out(nj)
                start_in(t + 3, nj)
        return carry

    lax.fori_loop(0, _TCH // _NBUF, outer, 0)
    for j in range(_NBUF):
        wait_out(j)


def kernel(patches, pos_table):
    flat = patches.reshape(_BATCH * _NP, _D)
    mesh = plsc.VectorSubcoreMesh(core_axis_name="c", subcore_axis_name="s")
    out = pl.kernel(
        _sc_body,
        out_type=jax.ShapeDtypeStruct((_BATCH * _NP, _D), jnp.float32),
        mesh=mesh,
        scratch_types=(
            [pltpu.VMEM((_BAND, _D), jnp.float32)]
            + [pltpu.VMEM((_C, _D), jnp.float32) for _ in range(_NBUF)]
            + [pltpu.SemaphoreType.DMA for _ in range(2 * _NBUF + 1)]
        ),
    )(flat, pos_table)
    return out.reshape(_BATCH, _NP, _D)


# speedup vs baseline: 1.0113x; 1.0113x over previous
import jax
import jax.numpy as jnp
from jax import lax
from jax.experimental import pallas as pl
from jax.experimental.pallas import tpu as pltpu
from jax.experimental.pallas import tpu_sc as plsc

_BATCH, _NP, _D = 64, 576, 768
_NC, _NS = 2, 16
_NW = _NC * _NS
_NG = 4
_NB = 8
_BPG = _BATCH // _NG
_BAND = _NP // _NB
_C = 24
_SPB = _BAND // _C
_TCH = _BPG * _SPB
_VPR = _D // 16
_NBUF = 4


def _sc_body(flat_hbm, pos_hbm, out_hbm, posband,
             b0, b1, b2, b3, si0, si1, si2, si3, so0, so1, so2, so3, sst):
    c = lax.axis_index("c")
    s = lax.axis_index("s")
    w = c * _NS + s
    g = w // _NB
    pb = w % _NB
    bufs = (b0, b1, b2, b3)
    sin = (si0, si1, si2, si3)
    sout = (so0, so1, so2, so3)

    stage = pltpu.async_copy(pos_hbm.at[pl.ds(pb * _BAND, _BAND)], posband, sst)

    def row0_of(t):
        b = g * _BPG + t // _SPB
        sub = t % _SPB
        return b * _NP + pb * _BAND + sub * _C

    def start_in(t, j):
        pltpu.async_copy(flat_hbm.at[pl.ds(row0_of(t), _C)], bufs[j], sin[j])

    def wait_in(j):
        pltpu.make_async_copy(flat_hbm.at[pl.ds(0, _C)], bufs[j], sin[j]).wait()

    def start_out(t, j):
        pltpu.async_copy(bufs[j], out_hbm.at[pl.ds(row0_of(t), _C)], sout[j])

    def wait_out(j):
        pltpu.make_async_copy(bufs[j], out_hbm.at[pl.ds(0, _C)], sout[j]).wait()

    start_in(0, 0)
    start_in(1, 1)
    start_in(2, 2)
    stage.wait()

    def outer(g2, carry):
        for j in range(_NBUF):
            t = g2 * _NBUF + j
            wait_in(j)
            prow = (t % _SPB) * _C

            @plsc.parallel_loop(0, _C)
            def row_add(i):
                for k in range(_VPR):
                    plsc.addupdate(bufs[j].at[i, pl.ds(k * 16, 16)],
                                   posband[prow + i, pl.ds(k * 16, 16)])
            start_out(t, j)
            nj = (j + 3) % _NBUF

            @pl.when(t + 3 < _TCH)
            def _prefetch():
                @pl.when(t >= 1)
                def _drain():
                    wait_out(nj)
                start_in(t + 3, nj)
        return carry

    lax.fori_loop(0, _TCH // _NBUF, outer, 0)
    for j in range(_NBUF):
        wait_out(j)


def kernel(patches, pos_table):
    flat = patches.reshape(_BATCH * _NP, _D)
    mesh = plsc.VectorSubcoreMesh(core_axis_name="c", subcore_axis_name="s")
    out = pl.kernel(
        _sc_body,
        out_type=jax.ShapeDtypeStruct((_BATCH * _NP, _D), jnp.float32),
        mesh=mesh,
        scratch_types=(
            [pltpu.VMEM((_BAND, _D), jnp.float32)]
            + [pltpu.VMEM((_C, _D), jnp.float32) for _ in range(_NBUF)]
            + [pltpu.SemaphoreType.DMA for _ in range(2 * _NBUF + 1)]
        ),
    )(flat, pos_table)
    return out.reshape(_BATCH, _NP, _D)
